# baseline (speedup 1.0000x reference)
import functools

import jax
import jax.numpy as jnp
from jax import lax
from jax.experimental import pallas as pl
from jax.experimental.pallas import tpu as pltpu
from jax.experimental.pallas import tpu_sc as plsc

_B, _S, _H, _D = 1024, 200, 8, 32
_NPH = 100000
_NB = 128
_NBT = _B // _NB
_IPW = _S // (32 // _NBT)
_PITCH = _D
_NBUF = 2

_mesh = plsc.VectorSubcoreMesh(core_axis_name="c", subcore_axis_name="s")


@functools.partial(
    pl.kernel,
    out_type=jax.ShapeDtypeStruct((_S, _H, _D // 8, _NBT, 8, _NB),
                                  jnp.float32),
    mesh=_mesh,
    compiler_params=pltpu.CompilerParams(
        use_tc_tiling_on_sc=False, needs_layout_passes=False),
    scratch_types=[
        pltpu.VMEM((_NBUF, _H, _NB), jnp.int32),
        pltpu.VMEM((_NBUF, _H, _NB, _PITCH), jnp.float32),
        pltpu.VMEM((2, _D // 8, 8, _NB), jnp.float32),
        pltpu.SemaphoreType.DMA,
        pltpu.SemaphoreType.DMA,
        pltpu.SemaphoreType.DMA,
        pltpu.SemaphoreType.DMA,
        pltpu.SemaphoreType.DMA,
        pltpu.SemaphoreType.DMA,
    ],
)
def _mhe_kernel(ids_hbm, table_hbm, out_hbm, raw_v, rows_v, obuf_v,
                sem_i0, sem_i1, sem_g0, sem_g1, sem_w0, sem_w1):
    wid = lax.axis_index("s") * 2 + lax.axis_index("c")
    bt = wid % _NBT
    b0 = bt * _NB
    s0 = (wid // _NBT) * _IPW

    sem_i = (sem_i0, sem_i1)
    sem_g = (sem_g0, sem_g1)
    sem_w = (sem_w0, sem_w1)

    iota = lax.iota(jnp.int32, 16)

    def idx_copy(item, slot):
        return pltpu.make_async_copy(
            ids_hbm.at[s0 + item, :, pl.ds(b0, _NB)],
            raw_v.at[slot], sem_i[slot])

    def write_copy(item, h, par):
        return pltpu.make_async_copy(
            obuf_v.at[par],
            out_hbm.at[s0 + item, h, :, bt], sem_w[par])

    def gather_drain(item, slot):
        @pl.loop(0, _H)
        def _h(h):
            pltpu.make_async_copy(
                out_hbm.at[s0 + item, h, :, bt],
                rows_v.at[slot, h], sem_g[slot]).wait()

    def stage_and_fire(item, slot):
        idx_copy(item, slot).wait()

        @pl.loop(0, _H)
        def _h(h):
            off = h * _NPH
            for j in range(_NB // 16):
                raw_v[slot, h, pl.ds(16 * j, 16)] = (
                    raw_v[slot, h, pl.ds(16 * j, 16)] + off)
            pltpu.make_async_copy(
                table_hbm.at[raw_v.at[slot, h]],
                rows_v.at[slot, h], sem_g[slot]).start()

    def transpose_and_write(item, slot):
        for h in range(_H):
            par = h % 2

            if h >= 2:
                write_copy(item, h - 2, par).wait()
            else:
                @pl.when(item >= 1)
                def _():
                    write_copy(item - 1, h + _H - 2, par).wait()

            rh = rows_v.at[slot, h]

            @pl.loop(0, _D, unroll=2)
            def _row(r):
                colv = jnp.full((16,), r, jnp.int32)
                for g in range(_NB // 16):
                    v = plsc.load_gather(rh, [iota + 16 * g, colv])
                    obuf_v[par, r >> 3, r & 7, pl.ds(16 * g, 16)] = v

            write_copy(item, h, par).start()

    idx_copy(0, 0).start()

    @pl.loop(0, _IPW, step=_NBUF)
    def _items(i0):
        for slot in range(_NBUF):
            it = i0 + slot
            oslot = (slot + 1) % _NBUF
            stage_and_fire(it, slot)

            @pl.when(it + 1 < _IPW)
            def _():
                idx_copy(it + 1, oslot).start()

            @pl.when(it >= 1)
            def _():
                transpose_and_write(it - 1, oslot)

            gather_drain(it, slot)

    transpose_and_write(_IPW - 1, (_IPW - 1) % _NBUF)
    write_copy(_IPW - 1, _H - 2, 0).wait()
    write_copy(_IPW - 1, _H - 1, 1).wait()


def kernel(input_ids, table):
    ids3 = jnp.transpose(input_ids, (1, 2, 0))
    out6 = _mhe_kernel(ids3, table)
    return jnp.transpose(out6, (0, 3, 5, 1, 2, 4)).reshape(_S, _B, _H, _D)


# speedup vs baseline: 1.2573x; 1.2573x over previous
import functools

import jax
import jax.numpy as jnp
from jax import lax
from jax.experimental import pallas as pl
from jax.experimental.pallas import tpu as pltpu
from jax.experimental.pallas import tpu_sc as plsc

_B, _S, _H, _D = 1024, 200, 8, 32
_NPH = 100000
_NB = 128
_NBT = _B // _NB
_IPW = _S // (32 // _NBT)
_PITCH = _D
_NBUF = 2

_mesh = plsc.VectorSubcoreMesh(core_axis_name="c", subcore_axis_name="s")


@functools.partial(
    pl.kernel,
    out_type=jax.ShapeDtypeStruct((_S, _H, _D // 8, _NBT, 8, _NB),
                                  jnp.float32),
    mesh=_mesh,
    compiler_params=pltpu.CompilerParams(
        use_tc_tiling_on_sc=False, needs_layout_passes=False),
    scratch_types=[
        pltpu.VMEM((_NBUF, _H, _NB), jnp.int32),
        pltpu.VMEM((_NBUF, _H, _NB, _PITCH), jnp.float32),
        pltpu.VMEM((2, _D // 8, 8, _NB), jnp.float32),
        pltpu.VMEM((_NB, _D + 1), jnp.float32),
        pltpu.SemaphoreType.DMA,
        pltpu.SemaphoreType.DMA,
        pltpu.SemaphoreType.DMA,
        pltpu.SemaphoreType.DMA,
        pltpu.SemaphoreType.DMA,
        pltpu.SemaphoreType.DMA,
    ],
)
def _mhe_kernel(ids_hbm, table_hbm, out_hbm, raw_v, rows_v, obuf_v, rskew_v,
                sem_i0, sem_i1, sem_g0, sem_g1, sem_w0, sem_w1):
    wid = lax.axis_index("s") * 2 + lax.axis_index("c")
    bt = wid % _NBT
    b0 = bt * _NB
    s0 = (wid // _NBT) * _IPW

    sem_i = (sem_i0, sem_i1)
    sem_g = (sem_g0, sem_g1)
    sem_w = (sem_w0, sem_w1)

    iota = lax.iota(jnp.int32, 16)

    def idx_copy(item, slot):
        return pltpu.make_async_copy(
            ids_hbm.at[s0 + item, :, pl.ds(b0, _NB)],
            raw_v.at[slot], sem_i[slot])

    def write_copy(item, h, par):
        return pltpu.make_async_copy(
            obuf_v.at[par],
            out_hbm.at[s0 + item, h, :, bt], sem_w[par])

    def gather_drain(item, slot):
        @pl.loop(0, _H)
        def _h(h):
            pltpu.make_async_copy(
                out_hbm.at[s0 + item, h, :, bt],
                rows_v.at[slot, h], sem_g[slot]).wait()

    def stage_and_fire(item, slot):
        idx_copy(item, slot).wait()

        @pl.loop(0, _H)
        def _h(h):
            off = h * _NPH
            for j in range(_NB // 16):
                raw_v[slot, h, pl.ds(16 * j, 16)] = (
                    raw_v[slot, h, pl.ds(16 * j, 16)] + off)
            pltpu.make_async_copy(
                table_hbm.at[raw_v.at[slot, h]],
                rows_v.at[slot, h], sem_g[slot]).start()

    def transpose_and_write(item, slot):
        for h in range(_H):
            par = h % 2

            if h >= 2:
                write_copy(item, h - 2, par).wait()
            else:
                @pl.when(item >= 1)
                def _():
                    write_copy(item - 1, h + _H - 2, par).wait()

            @pl.loop(0, _NB, unroll=4)
            def _b(b):
                rskew_v[b, pl.ds(0, 16)] = rows_v[slot, h, b, pl.ds(0, 16)]
                rskew_v[b, pl.ds(16, 16)] = rows_v[slot, h, b, pl.ds(16, 16)]

            @pl.loop(0, _D, unroll=2)
            def _row(r):
                colv = jnp.full((16,), r, jnp.int32)
                for g in range(_NB // 16):
                    v = plsc.load_gather(rskew_v, [iota + 16 * g, colv])
                    obuf_v[par, r >> 3, r & 7, pl.ds(16 * g, 16)] = v

            write_copy(item, h, par).start()

    idx_copy(0, 0).start()

    @pl.loop(0, _IPW, step=_NBUF)
    def _items(i0):
        for slot in range(_NBUF):
            it = i0 + slot
            oslot = (slot + 1) % _NBUF
            stage_and_fire(it, slot)

            @pl.when(it + 1 < _IPW)
            def _():
                idx_copy(it + 1, oslot).start()

            @pl.when(it >= 1)
            def _():
                transpose_and_write(it - 1, oslot)

            gather_drain(it, slot)

    transpose_and_write(_IPW - 1, (_IPW - 1) % _NBUF)
    write_copy(_IPW - 1, _H - 2, 0).wait()
    write_copy(_IPW - 1, _H - 1, 1).wait()


def kernel(input_ids, table):
    ids3 = jnp.transpose(input_ids, (1, 2, 0))
    out6 = _mhe_kernel(ids3, table)
    return jnp.transpose(out6, (0, 3, 5, 1, 2, 4)).reshape(_S, _B, _H, _D)


# speedup vs baseline: 1.2587x; 1.0011x over previous
import functools

import jax
import jax.numpy as jnp
from jax import lax
from jax.experimental import pallas as pl
from jax.experimental.pallas import tpu as pltpu
from jax.experimental.pallas import tpu_sc as plsc

_B, _S, _H, _D = 1024, 200, 8, 32
_NPH = 100000
_NB = 128
_NBT = _B // _NB
_IPW = _S // (32 // _NBT)
_PITCH = _D
_NBUF = 2

_mesh = plsc.VectorSubcoreMesh(core_axis_name="c", subcore_axis_name="s")


@functools.partial(
    pl.kernel,
    out_type=jax.ShapeDtypeStruct((_S, _H, _D // 8, _NBT, 8, _NB),
                                  jnp.float32),
    mesh=_mesh,
    compiler_params=pltpu.CompilerParams(
        use_tc_tiling_on_sc=False, needs_layout_passes=False),
    scratch_types=[
        pltpu.VMEM((_NBUF, _H, _NB), jnp.int32),
        pltpu.VMEM((_NBUF, _H, _NB, _PITCH), jnp.float32),
        pltpu.VMEM((2, _D // 8, 8, _NB), jnp.float32),
        pltpu.VMEM((_NB, _D + 1), jnp.float32),
        pltpu.SemaphoreType.DMA,
        pltpu.SemaphoreType.DMA,
        pltpu.SemaphoreType.DMA,
        pltpu.SemaphoreType.DMA,
        pltpu.SemaphoreType.DMA,
        pltpu.SemaphoreType.DMA,
    ],
)
def _mhe_kernel(ids_hbm, table_hbm, out_hbm, raw_v, rows_v, obuf_v, rskew_v,
                sem_i0, sem_i1, sem_g0, sem_g1, sem_w0, sem_w1):
    wid = lax.axis_index("s") * 2 + lax.axis_index("c")
    bt = wid % _NBT
    b0 = bt * _NB
    s0 = (wid // _NBT) * _IPW

    sem_i = (sem_i0, sem_i1)
    sem_g = (sem_g0, sem_g1)
    sem_w = (sem_w0, sem_w1)

    iota = lax.iota(jnp.int32, 16)

    def idx_copy(item, slot):
        return pltpu.make_async_copy(
            ids_hbm.at[s0 + item, :, pl.ds(b0, _NB)],
            raw_v.at[slot], sem_i[slot])

    def write_copy(item, h, par):
        return pltpu.make_async_copy(
            obuf_v.at[par],
            out_hbm.at[s0 + item, h, :, bt], sem_w[par])

    def gather_drain(item, slot):
        @pl.loop(0, _H)
        def _h(h):
            pltpu.make_async_copy(
                out_hbm.at[s0 + item, h, :, bt],
                rows_v.at[slot, h], sem_g[slot]).wait()

    def stage_and_fire(item, slot):
        idx_copy(item, slot).wait()

        @pl.loop(0, _H)
        def _h(h):
            off = h * _NPH
            for j in range(_NB // 16):
                raw_v[slot, h, pl.ds(16 * j, 16)] = (
                    raw_v[slot, h, pl.ds(16 * j, 16)] + off)
            pltpu.make_async_copy(
                table_hbm.at[raw_v.at[slot, h]],
                rows_v.at[slot, h], sem_g[slot]).start()

    def transpose_and_write(item, slot):
        for h in range(_H):
            par = h % 2

            if h >= 2:
                write_copy(item, h - 2, par).wait()
            else:
                @pl.when(item >= 1)
                def _():
                    write_copy(item - 1, h + _H - 2, par).wait()

            @pl.loop(0, _NB, unroll=8)
            def _b(b):
                rskew_v[b, pl.ds(0, 16)] = rows_v[slot, h, b, pl.ds(0, 16)]
                rskew_v[b, pl.ds(16, 16)] = rows_v[slot, h, b, pl.ds(16, 16)]

            @pl.loop(0, _D, unroll=2)
            def _row(r):
                colv = jnp.full((16,), r, jnp.int32)
                for g in range(_NB // 16):
                    v = plsc.load_gather(rskew_v, [iota + 16 * g, colv])
                    obuf_v[par, r >> 3, r & 7, pl.ds(16 * g, 16)] = v

            write_copy(item, h, par).start()

    idx_copy(0, 0).start()

    @pl.loop(0, _IPW, step=_NBUF)
    def _items(i0):
        for slot in range(_NBUF):
            it = i0 + slot
            oslot = (slot + 1) % _NBUF
            stage_and_fire(it, slot)

            @pl.when(it + 1 < _IPW)
            def _():
                idx_copy(it + 1, oslot).start()

            @pl.when(it >= 1)
            def _():
                transpose_and_write(it - 1, oslot)

            gather_drain(it, slot)

    transpose_and_write(_IPW - 1, (_IPW - 1) % _NBUF)
    write_copy(_IPW - 1, _H - 2, 0).wait()
    write_copy(_IPW - 1, _H - 1, 1).wait()


def kernel(input_ids, table):
    ids3 = jnp.transpose(input_ids, (1, 2, 0))
    out6 = _mhe_kernel(ids3, table)
    return jnp.transpose(out6, (0, 3, 5, 1, 2, 4)).reshape(_S, _B, _H, _D)
